# Initial kernel scaffold; baseline (speedup 1.0000x reference)
import functools

import jax
import jax.numpy as jnp
from jax import lax
from jax.experimental import pallas as pl
from jax.experimental.pallas import tpu as pltpu
from jax.experimental.pallas import tpu_sc as plsc

N_ART = 10000
N_STYLE = 10000
E = 320000
D = 128
C = 64

NC = 2
NS = 16
NW = NC * NS
EPW = E // NW
CH = 80
NCH = EPW // CH
RPT = N_STYLE // NS
ZR = 125

_LANES = 16


def _proj_body(xa_ref, xs_ref, ws_ref, wd_ref, av_ref, bv_ref,
               h_ref, as_ref, ad_ref):
    h = jnp.dot(xa_ref[...], ws_ref[...], preferred_element_type=jnp.float32)
    h_ref[...] = h
    as_ref[...] = jnp.sum(h * av_ref[...][None, :], axis=1)
    hd = jnp.dot(xs_ref[...], wd_ref[...], preferred_element_type=jnp.float32)
    ad_ref[...] = jnp.sum(hd * bv_ref[...][None, :], axis=1)


def _project(x_artwork, x_style, W_src, W_dst, a_src, a_dst):
    bm = 1000
    grid = N_ART // bm
    return pl.pallas_call(
        _proj_body,
        grid=(grid,),
        in_specs=[
            pl.BlockSpec((bm, D), lambda i: (i, 0)),
            pl.BlockSpec((bm, D), lambda i: (i, 0)),
            pl.BlockSpec((D, C), lambda i: (0, 0)),
            pl.BlockSpec((D, C), lambda i: (0, 0)),
            pl.BlockSpec((C,), lambda i: (0,)),
            pl.BlockSpec((C,), lambda i: (0,)),
        ],
        out_specs=[
            pl.BlockSpec((bm, C), lambda i: (i, 0)),
            pl.BlockSpec((bm,), lambda i: (i,)),
            pl.BlockSpec((bm,), lambda i: (i,)),
        ],
        out_shape=[
            jax.ShapeDtypeStruct((N_ART, C), jnp.float32),
            jax.ShapeDtypeStruct((N_ART,), jnp.float32),
            jax.ShapeDtypeStruct((N_STYLE,), jnp.float32),
        ],
    )(x_artwork, x_style, W_src, W_dst, a_src, a_dst)


def _edge_body(src2_hbm, dst2_hbm, srcf_hbm, dstf_hbm, asrc_hbm, adst_hbm,
               h_hbm, acc_out, den_out,
               srcv2, dstv2, srcf, dstf, asv, adv, denv, rowsv, exv, zbuf,
               acc_sh, gsem):
    cid = lax.axis_index("c")
    sid = lax.axis_index("s")
    wid = cid * NS + sid

    pltpu.sync_copy(src2_hbm.at[pl.ds(wid * NCH, NCH)], srcv2)
    pltpu.sync_copy(dst2_hbm.at[pl.ds(wid * NCH, NCH)], dstv2)
    pltpu.sync_copy(srcf_hbm.at[pl.ds(wid * EPW, EPW)], srcf)
    pltpu.sync_copy(dstf_hbm.at[pl.ds(wid * EPW, EPW)], dstf)
    pltpu.sync_copy(asrc_hbm, asv)
    pltpu.sync_copy(adst_hbm, adv)

    z16 = jnp.zeros((_LANES,), jnp.float32)

    def zden(i, c):
        denv[pl.ds(i * _LANES, _LANES)] = z16
        return c
    lax.fori_loop(0, N_STYLE // _LANES, zden, 0)

    def zbody(i, c):
        for k in range(C // _LANES):
            zbuf[i, pl.ds(k * _LANES, _LANES)] = z16
        return c
    lax.fori_loop(0, ZR, zbody, 0)

    for j in range(RPT // ZR):
        pltpu.sync_copy(zbuf, acc_sh.at[pl.ds(sid * RPT + j * ZR, ZR)])
    plsc.subcore_barrier()

    def chunk(c, carry):
        cp = pltpu.async_copy(h_hbm.at[srcv2.at[c]], rowsv, gsem)
        base = c * CH
        for k in range(CH // _LANES):
            sv = srcf[pl.ds(base + k * _LANES, _LANES)]
            dv = dstf[pl.ds(base + k * _LANES, _LANES)]
            a = plsc.load_gather(asv, [sv]) + plsc.load_gather(adv, [dv])
            e = jnp.where(a >= 0, a, 0.2 * a)
            ex = jnp.exp(e)
            exv[pl.ds(k * _LANES, _LANES)] = ex
            plsc.addupdate_scatter(denv, [dv], ex)
        cp.wait()
        for j in range(CH):
            b = plsc.load_gather(exv, [jnp.full((_LANES,), j, jnp.int32)])
            for k in range(C // _LANES):
                sl = pl.ds(k * _LANES, _LANES)
                rowsv[j, sl] = rowsv[j, sl] * b
        pltpu.sync_copy(rowsv, acc_sh.at[dstv2.at[c]], add=True)
        return carry

    lax.fori_loop(0, NCH, chunk, 0)

    pltpu.sync_copy(denv, den_out.at[wid])

    plsc.subcore_barrier()
    for j in range(RPT // ZR):
        rs = pl.ds(sid * RPT + j * ZR, ZR)
        pltpu.sync_copy(acc_sh.at[rs], zbuf)
        pltpu.sync_copy(zbuf, acc_out.at[cid].at[rs])


def _edge_phase(src2, dst2, srcf, dstf, asrc, adst, h):
    mesh = plsc.VectorSubcoreMesh(core_axis_name="c", subcore_axis_name="s")
    fn = functools.partial(
        pl.kernel,
        out_type=[
            jax.ShapeDtypeStruct((NC, N_STYLE, C), jnp.float32),
            jax.ShapeDtypeStruct((NW, N_STYLE), jnp.float32),
        ],
        mesh=mesh,
        scratch_types=[
            pltpu.VMEM((NCH, CH), jnp.int32),
            pltpu.VMEM((NCH, CH), jnp.int32),
            pltpu.VMEM((EPW,), jnp.int32),
            pltpu.VMEM((EPW,), jnp.int32),
            pltpu.VMEM((N_ART,), jnp.float32),
            pltpu.VMEM((N_STYLE,), jnp.float32),
            pltpu.VMEM((N_STYLE,), jnp.float32),
            pltpu.VMEM((CH, C), jnp.float32),
            pltpu.VMEM((CH,), jnp.float32),
            pltpu.VMEM((ZR, C), jnp.float32),
            pltpu.VMEM_SHARED((N_STYLE, C), jnp.float32),
            pltpu.SemaphoreType.DMA,
        ],
    )(_edge_body)
    return fn(src2, dst2, srcf, dstf, asrc, adst, h)


def _epi_body(acc_ref, den_ref, bias_ref, out_ref):
    den = jnp.sum(den_ref[...], axis=0)
    num = acc_ref[0] + acc_ref[1]
    out = num / (den[:, None] + 1e-16) + bias_ref[...][None, :]
    out_ref[...] = jnp.maximum(out, 0.0)


def _epilogue(acc, den, bias):
    bm = 1000
    grid = N_STYLE // bm
    return pl.pallas_call(
        _epi_body,
        grid=(grid,),
        in_specs=[
            pl.BlockSpec((NC, bm, C), lambda i: (0, i, 0)),
            pl.BlockSpec((NW, bm), lambda i: (0, i)),
            pl.BlockSpec((C,), lambda i: (0,)),
        ],
        out_specs=pl.BlockSpec((bm, C), lambda i: (i, 0)),
        out_shape=jax.ShapeDtypeStruct((N_STYLE, C), jnp.float32),
    )(acc, den, bias)


def kernel(x_artwork, x_style, edge_index, W_src, W_dst, a_src, a_dst, bias):
    src = edge_index[0].astype(jnp.int32)
    dst = edge_index[1].astype(jnp.int32)
    src2 = src.reshape(E // CH, CH)
    dst2 = dst.reshape(E // CH, CH)

    h, asrc, adst = _project(x_artwork, x_style, W_src, W_dst, a_src, a_dst)
    acc, den = _edge_phase(src2, dst2, src, dst, asrc, adst, h)
    out = _epilogue(acc, den, bias)
    return out.reshape(-1)


# speedup vs baseline: 37.3607x; 37.3607x over previous
import functools

import jax
import jax.numpy as jnp
from jax import lax
from jax.experimental import pallas as pl
from jax.experimental.pallas import tpu as pltpu
from jax.experimental.pallas import tpu_sc as plsc

N_ART = 10000
N_STYLE = 10000
E = 320000
D = 128
C = 64

NC = 2
NS = 16
NW = NC * NS
EPW = E // NW
CH = 80
NCH = EPW // CH
RPT = N_STYLE // NS
ZR = 125

_LANES = 16


def _proj_body(xa_ref, xs_ref, ws_ref, wd_ref, av_ref, bv_ref,
               h_ref, as_ref, ad_ref):
    h = jnp.dot(xa_ref[...], ws_ref[...], preferred_element_type=jnp.float32)
    h_ref[...] = h
    as_ref[...] = jnp.sum(h * av_ref[...][None, :], axis=1)[None, None, :]
    hd = jnp.dot(xs_ref[...], wd_ref[...], preferred_element_type=jnp.float32)
    ad_ref[...] = jnp.sum(hd * bv_ref[...][None, :], axis=1)[None, None, :]


def _project(x_artwork, x_style, W_src, W_dst, a_src, a_dst):
    bm = 1000
    grid = N_ART // bm
    return pl.pallas_call(
        _proj_body,
        grid=(grid,),
        in_specs=[
            pl.BlockSpec((bm, D), lambda i: (i, 0)),
            pl.BlockSpec((bm, D), lambda i: (i, 0)),
            pl.BlockSpec((D, C), lambda i: (0, 0)),
            pl.BlockSpec((D, C), lambda i: (0, 0)),
            pl.BlockSpec((C,), lambda i: (0,)),
            pl.BlockSpec((C,), lambda i: (0,)),
        ],
        out_specs=[
            pl.BlockSpec((bm, C), lambda i: (i, 0)),
            pl.BlockSpec((1, 1, bm), lambda i: (i, 0, 0)),
            pl.BlockSpec((1, 1, bm), lambda i: (i, 0, 0)),
        ],
        out_shape=[
            jax.ShapeDtypeStruct((N_ART, C), jnp.float32),
            jax.ShapeDtypeStruct((N_ART // bm, 1, bm), jnp.float32),
            jax.ShapeDtypeStruct((N_STYLE // bm, 1, bm), jnp.float32),
        ],
    )(x_artwork, x_style, W_src, W_dst, a_src, a_dst)


def _edge_body(src2_hbm, dst2_hbm, asrc_hbm, adst_hbm,
               h_hbm, acc_out, den_out,
               srcv2, dstv2, asv, adv, denv, rowsv, exv, zbuf,
               acc_sh, gsem):
    cid = lax.axis_index("c")
    sid = lax.axis_index("s")
    wid = cid * NS + sid

    pltpu.sync_copy(src2_hbm.at[pl.ds(wid * NCH, NCH)], srcv2)
    pltpu.sync_copy(dst2_hbm.at[pl.ds(wid * NCH, NCH)], dstv2)
    pltpu.sync_copy(asrc_hbm, asv)
    pltpu.sync_copy(adst_hbm, adv)

    z16 = jnp.zeros((_LANES,), jnp.float32)

    def zden(i, c):
        denv[pl.ds(i * _LANES, _LANES)] = z16
        return c
    lax.fori_loop(0, N_STYLE // _LANES, zden, 0)

    def zbody(i, c):
        for k in range(C // _LANES):
            zbuf[i, pl.ds(k * _LANES, _LANES)] = z16
        return c
    lax.fori_loop(0, ZR, zbody, 0)

    for j in range(RPT // ZR):
        pltpu.sync_copy(zbuf, acc_sh.at[pl.ds(sid * RPT + j * ZR, ZR)])
    plsc.subcore_barrier()

    def chunk(c, carry):
        cp = pltpu.async_copy(h_hbm.at[srcv2.at[c]], rowsv, gsem)
        for k in range(CH // _LANES):
            sv = srcv2[c, pl.ds(k * _LANES, _LANES)]
            dv = dstv2[c, pl.ds(k * _LANES, _LANES)]
            a = plsc.load_gather(asv, [sv]) + plsc.load_gather(adv, [dv])
            e = jnp.where(a >= 0, a, 0.2 * a)
            ex = jnp.exp(e)
            exv[pl.ds(k * _LANES, _LANES)] = ex
            plsc.addupdate_scatter(denv, [dv], ex)
        cp.wait()
        for j in range(CH):
            b = plsc.load_gather(exv, [jnp.full((_LANES,), j, jnp.int32)])
            for k in range(C // _LANES):
                sl = pl.ds(k * _LANES, _LANES)
                rowsv[j, sl] = rowsv[j, sl] * b
        pltpu.sync_copy(rowsv, acc_sh.at[dstv2.at[c]], add=True)
        return carry

    lax.fori_loop(0, NCH, chunk, 0)

    pltpu.sync_copy(denv, den_out.at[wid])

    plsc.subcore_barrier()
    for j in range(RPT // ZR):
        rs = pl.ds(sid * RPT + j * ZR, ZR)
        pltpu.sync_copy(acc_sh.at[rs], zbuf)
        pltpu.sync_copy(zbuf, acc_out.at[cid].at[rs])


def _edge_phase(src2, dst2, asrc, adst, h):
    mesh = plsc.VectorSubcoreMesh(core_axis_name="c", subcore_axis_name="s")
    fn = functools.partial(
        pl.kernel,
        out_type=[
            jax.ShapeDtypeStruct((NC, N_STYLE, C), jnp.float32),
            jax.ShapeDtypeStruct((NW, N_STYLE), jnp.float32),
        ],
        mesh=mesh,
        scratch_types=[
            pltpu.VMEM((NCH, CH), jnp.int32),
            pltpu.VMEM((NCH, CH), jnp.int32),
            pltpu.VMEM((N_ART,), jnp.float32),
            pltpu.VMEM((N_STYLE,), jnp.float32),
            pltpu.VMEM((N_STYLE,), jnp.float32),
            pltpu.VMEM((CH, C), jnp.float32),
            pltpu.VMEM((CH,), jnp.float32),
            pltpu.VMEM((ZR, C), jnp.float32),
            pltpu.VMEM_SHARED((N_STYLE, C), jnp.float32),
            pltpu.SemaphoreType.DMA,
        ],
        compiler_params=pltpu.CompilerParams(
            use_tc_tiling_on_sc=False, needs_layout_passes=False),
    )(_edge_body)
    return fn(src2, dst2, asrc, adst, h)


def _epi_body(acc_ref, den_ref, bias_ref, out_ref):
    den = jnp.sum(den_ref[...], axis=0)
    num = acc_ref[0] + acc_ref[1]
    out = num / (den[:, None] + 1e-16) + bias_ref[...][None, :]
    out_ref[...] = jnp.maximum(out, 0.0)


def _epilogue(acc, den, bias):
    return pl.pallas_call(
        _epi_body,
        out_shape=jax.ShapeDtypeStruct((N_STYLE, C), jnp.float32),
    )(acc, den, bias)


def kernel(x_artwork, x_style, edge_index, W_src, W_dst, a_src, a_dst, bias):
    src = edge_index[0].astype(jnp.int32)
    dst = edge_index[1].astype(jnp.int32)
    src2 = src.reshape(E // CH, CH)
    dst2 = dst.reshape(E // CH, CH)

    h, asrc, adst = _project(x_artwork, x_style, W_src, W_dst, a_src, a_dst)
    asrc = asrc.reshape(-1)
    adst = adst.reshape(-1)
    acc, den = _edge_phase(src2, dst2, asrc, adst, h)
    out = _epilogue(acc, den, bias)
    return out.reshape(-1)
